# Initial kernel scaffold; baseline (speedup 1.0000x reference)
import jax
import jax.numpy as jnp
from jax.experimental import pallas as pl


def kernel(img_batch, features, img_sizes, og_sizes, rpn_conv_w, rpn_conv_b, rpn_cls_w, rpn_cls_b, rpn_bbox_w, rpn_bbox_b, fc1_w, fc1_b, fc2_w, fc2_b, cls_w, cls_b, bbox_w, bbox_b):
    raise NotImplementedError("write your pallas kernel here")


# speedup vs baseline: 5.2565x; 5.2565x over previous
import functools

import jax
import jax.numpy as jnp
import numpy as np
from jax.experimental import pallas as pl

_B = 2
_C = 256
_H = 50
_W = 50
_A = 3
_STRIDE = 16
_IMG = 800
_PRE = 1000
_POST = 300
_DETS = 100
_NCLS = 91
_POOL = 7

_NANCH = _A * _H * _W
_NPAD = 8192
_NROW = _NPAD // 128
_NB = 304
_HWPAD = 2560
_LOGMAX = float(np.log(1000.0 / 16.0))
_XCLIP = np.float32(_W - 1.001)


def _rpn_kernel(x_ref, wc_ref, bc_ref, wh_ref, bh_ref, o_ref):
    x = x_ref[0]
    t = jnp.maximum(
        jnp.dot(x, wc_ref[...], preferred_element_type=jnp.float32)
        + bc_ref[0:1, :], 0.0)
    o_ref[0] = (jnp.dot(t, wh_ref[...], preferred_element_type=jnp.float32)
                + bh_ref[0:1, :])


def _nms1_kernel(obj_ref, dl_ref, an_ref, pb_ref):
    o = obj_ref[0]
    bits = jax.lax.bitcast_convert_type(o, jnp.int32)
    skey = bits ^ (jax.lax.shift_right_arithmetic(bits, 31) & 0x7FFFFFFF)

    cnt0 = jnp.sum((skey >= 0).astype(jnp.int32))
    t0 = jnp.where(cnt0 >= _PRE, jnp.int32(0), jnp.int32(-2147483648))

    def bitstep(i, t):
        t_try = t + jax.lax.shift_left(jnp.int32(1), 30 - i)
        cnt = jnp.sum((skey >= t_try).astype(jnp.int32))
        return jnp.where(cnt >= _PRE, t_try, t)

    t = jax.lax.fori_loop(0, 31, bitstep, t0)
    cand = skey >= t

    sig = 1.0 / (1.0 + jnp.exp(-o))
    sc0 = jnp.where(cand, sig, -2e9)

    ax1 = an_ref[0]
    ay1 = an_ref[1]
    ax2 = an_ref[2]
    ay2 = an_ref[3]
    ddx = dl_ref[0, 0]
    ddy = dl_ref[0, 1]
    ddw = dl_ref[0, 2]
    ddh = dl_ref[0, 3]
    aw = ax2 - ax1
    ah = ay2 - ay1
    acx = ax1 + 0.5 * aw
    acy = ay1 + 0.5 * ah
    dwc = jnp.minimum(ddw, _LOGMAX)
    dhc = jnp.minimum(ddh, _LOGMAX)
    pcx = ddx * aw + acx
    pcy = ddy * ah + acy
    pw = jnp.exp(dwc) * aw
    ph = jnp.exp(dhc) * ah
    x1 = jnp.clip(pcx - 0.5 * pw, 0.0, float(_IMG))
    y1 = jnp.clip(pcy - 0.5 * ph, 0.0, float(_IMG))
    x2 = jnp.clip(pcx + 0.5 * pw, 0.0, float(_IMG))
    y2 = jnp.clip(pcy + 0.5 * ph, 0.0, float(_IMG))
    area = (x2 - x1) * (y2 - y1)

    iota = jax.lax.broadcasted_iota(jnp.int32, (_NROW, 128), 0) * 128 + \
        jax.lax.broadcasted_iota(jnp.int32, (_NROW, 128), 1)
    lane = jax.lax.broadcasted_iota(jnp.int32, (1, 128), 1)

    pb_ref[0, _POST:_NB, :] = jnp.zeros((_NB - _POST, 128), jnp.float32)

    def step(n, sc):
        m = jnp.max(sc)
        i = jnp.min(jnp.where(sc == m, iota, jnp.int32(2 ** 30)))
        oh = iota == i
        bx1 = jnp.sum(jnp.where(oh, x1, 0.0))
        by1 = jnp.sum(jnp.where(oh, y1, 0.0))
        bx2 = jnp.sum(jnp.where(oh, x2, 0.0))
        by2 = jnp.sum(jnp.where(oh, y2, 0.0))
        barea = (bx2 - bx1) * (by2 - by1)
        iw = jnp.maximum(jnp.minimum(x2, bx2) - jnp.maximum(x1, bx1), 0.0)
        ih = jnp.maximum(jnp.minimum(y2, by2) - jnp.maximum(y1, by1), 0.0)
        inter = iw * ih
        iou = inter / (area + barea - inter + 1e-6)
        sup = (iou >= 0.7) & (sc > -1.5e9)
        sc = jnp.where(sup, -1e9 + 1e-10 * sig, sc)
        row = jnp.where(lane == 0, bx1,
                        jnp.where(lane == 1, by1,
                                  jnp.where(lane == 2, bx2,
                                            jnp.where(lane == 3, by2, 0.0))))
        pb_ref[0, pl.ds(n, 1), :] = row
        return sc

    jax.lax.fori_loop(0, _POST, step, sc0, unroll=False)


def _roi_fc1_kernel(pb_ref, f_ref, w1_ref, b1_ref, cy_ref, cx_ref, o_ref):
    k = pl.program_id(1)
    ky = k // _POOL
    kx = k - ky * _POOL
    gy = (ky.astype(jnp.float32) + 0.5) / _POOL
    gx = (kx.astype(jnp.float32) + 0.5) / _POOL

    pb = pb_ref[0]
    bx1 = pb[:, 0:1] * (1.0 / _STRIDE)
    by1 = pb[:, 1:2] * (1.0 / _STRIDE)
    bx2 = pb[:, 2:3] * (1.0 / _STRIDE)
    by2 = pb[:, 3:4] * (1.0 / _STRIDE)
    xs = jnp.clip(bx1 + gx * (bx2 - bx1), 0.0, _XCLIP)
    ys = jnp.clip(by1 + gy * (by2 - by1), 0.0, _XCLIP)
    x0 = jnp.floor(xs)
    y0 = jnp.floor(ys)
    wx = xs - x0
    wy = ys - y0
    x1 = jnp.minimum(x0 + 1.0, float(_W - 1))
    y1 = jnp.minimum(y0 + 1.0, float(_H - 1))

    acc = jnp.zeros((_NB, _C), jnp.float32)
    for j in range(_HWPAD // 128):
        cy = cy_ref[0:1, j * 128:(j + 1) * 128]
        cx = cx_ref[0:1, j * 128:(j + 1) * 128]
        yterm = jnp.where(cy == y0, 1.0 - wy, 0.0) + \
            jnp.where(cy == y1, wy, 0.0)
        xterm = jnp.where(cx == x0, 1.0 - wx, 0.0) + \
            jnp.where(cx == x1, wx, 0.0)
        m = yterm * xterm
        acc = acc + jax.lax.dot_general(
            m, f_ref[0, j * 128:(j + 1) * 128, :], (((1,), (0,)), ((), ())),
            precision=jax.lax.Precision.HIGHEST,
            preferred_element_type=jnp.float32)

    accr = acc.astype(jnp.bfloat16).astype(jnp.float32)
    contrib = jnp.dot(accr, w1_ref[0], preferred_element_type=jnp.float32)

    @pl.when(k == 0)
    def _():
        o_ref[0] = contrib + b1_ref[0:1, :]

    @pl.when(k > 0)
    def _():
        o_ref[0] = o_ref[0] + contrib

    @pl.when(k == _POOL * _POOL - 1)
    def _():
        o_ref[0] = jnp.maximum(o_ref[0], 0.0)


def _head_kernel(h1_ref, w2_ref, b2_ref, wcl_ref, bcl_ref, wbb_ref, bbb_ref,
                 rc_ref, pb_ref, sr_ref, o_ref):
    h1 = h1_ref[0].astype(jnp.bfloat16).astype(jnp.float32)
    h2 = jnp.maximum(
        jnp.dot(h1, w2_ref[...], preferred_element_type=jnp.float32)
        + b2_ref[0:1, :], 0.0)
    h2b = h2.astype(jnp.bfloat16).astype(jnp.float32)
    cl = (jnp.dot(h2b, wcl_ref[...], preferred_element_type=jnp.float32)
          + bcl_ref[0:1, :])
    mx = jnp.max(cl, axis=1, keepdims=True)
    e = jnp.exp(cl - mx)
    pr = e / jnp.sum(e, axis=1, keepdims=True)
    lane = jax.lax.broadcasted_iota(jnp.int32, (_NB, 128), 1)
    fg = jnp.where((lane >= 1) & (lane <= _NCLS - 1), pr, -1.0)
    score = jnp.max(fg, axis=1, keepdims=True)
    label = jnp.min(jnp.where(fg == score, lane, jnp.int32(999)),
                    axis=1, keepdims=True)

    bb = (jnp.dot(h2b, wbb_ref[...], preferred_element_type=jnp.float32)
          + bbb_ref[0:1, :])
    l384 = jax.lax.broadcasted_iota(jnp.int32, (_NB, 384), 1)
    oh = jax.lax.shift_right_logical(l384, 2) == label
    d = jax.lax.dot_general(jnp.where(oh, bb, 0.0), rc_ref[...],
                            (((1,), (0,)), ((), ())),
                            precision=jax.lax.Precision.HIGHEST,
                            preferred_element_type=jnp.float32)
    ddx = d[:, 0:1]
    ddy = d[:, 1:2]
    ddw = jnp.minimum(d[:, 2:3], _LOGMAX)
    ddh = jnp.minimum(d[:, 3:4], _LOGMAX)

    pb = pb_ref[0]
    px1 = pb[:, 0:1]
    py1 = pb[:, 1:2]
    px2 = pb[:, 2:3]
    py2 = pb[:, 3:4]
    w = px2 - px1
    h = py2 - py1
    cx = px1 + 0.5 * w
    cy = py1 + 0.5 * h
    pcx = ddx * w + cx
    pcy = ddy * h + cy
    pw = jnp.exp(ddw) * w
    ph = jnp.exp(ddh) * h
    fx1 = jnp.clip(pcx - 0.5 * pw, 0.0, float(_IMG))
    fy1 = jnp.clip(pcy - 0.5 * ph, 0.0, float(_IMG))
    fx2 = jnp.clip(pcx + 0.5 * pw, 0.0, float(_IMG))
    fy2 = jnp.clip(pcy + 0.5 * ph, 0.0, float(_IMG))
    area = (fx2 - fx1) * (fy2 - fy1)

    rowi = jax.lax.broadcasted_iota(jnp.int32, (_NB, 1), 0)
    rowf = rowi.astype(jnp.float32)
    valid = rowi < _POST
    score_o = jnp.where(valid, score, 0.0)
    sc0 = jnp.where(valid, score, -2e9)
    lane1 = jax.lax.broadcasted_iota(jnp.int32, (1, 128), 1)
    srow = sr_ref[0, 0:1, :]

    def step(n, sc):
        m = jnp.max(sc)
        i = jnp.min(jnp.where(sc == m, rowi, jnp.int32(2 ** 30)))
        oh2 = rowi == i
        bx1 = jnp.sum(jnp.where(oh2, fx1, 0.0))
        by1 = jnp.sum(jnp.where(oh2, fy1, 0.0))
        bx2 = jnp.sum(jnp.where(oh2, fx2, 0.0))
        by2 = jnp.sum(jnp.where(oh2, fy2, 0.0))
        bsc = jnp.sum(jnp.where(oh2, score_o, 0.0))
        barea = (bx2 - bx1) * (by2 - by1)
        iw = jnp.maximum(jnp.minimum(fx2, bx2) - jnp.maximum(fx1, bx1), 0.0)
        ih = jnp.maximum(jnp.minimum(fy2, by2) - jnp.maximum(fy1, by1), 0.0)
        inter = iw * ih
        iou = inter / (area + barea - inter + 1e-6)
        sup = (iou >= 0.5) & (sc > -1.5e9)
        sc = jnp.where(sup, -1e9 - rowf, sc)
        row = jnp.where(lane1 == 0, bx1,
                        jnp.where(lane1 == 1, by1,
                                  jnp.where(lane1 == 2, bx2,
                                            jnp.where(lane1 == 3, by2,
                                                      jnp.where(lane1 == 4,
                                                                bsc, 0.0)))))
        o_ref[0, pl.ds(n, 1), :] = row * srow
        return sc

    jax.lax.fori_loop(0, _DETS, step, sc0, unroll=False)


def _build_anchors():
    sizes = np.array([64.0, 128.0, 256.0], np.float32)
    cx = np.arange(_W, dtype=np.float32) * _STRIDE
    cy = np.arange(_H, dtype=np.float32) * _STRIDE
    CY, CX = np.meshgrid(cy, cx, indexing='ij')
    half = sizes / 2.0
    x1 = CX[None] - half[:, None, None]
    y1 = CY[None] - half[:, None, None]
    x2 = CX[None] + half[:, None, None]
    y2 = CY[None] + half[:, None, None]
    a = np.stack([x1, y1, x2, y2], -1).reshape(-1, 4)
    ap = np.zeros((_NPAD, 4), np.float32)
    ap[:_NANCH] = a
    return ap.T.reshape(4, _NROW, 128)

_ANCHORS = jnp.asarray(_build_anchors())

_COL = np.arange(_HWPAD)
_COLY = jnp.asarray(np.broadcast_to(
    np.where(_COL < _H * _W, _COL // _W, -1).astype(np.float32), (8, _HWPAD)))
_COLX = jnp.asarray(np.broadcast_to(
    np.where(_COL < _H * _W, _COL % _W, -1).astype(np.float32), (8, _HWPAD)))

_RC = jnp.asarray(np.stack([(np.arange(384) % 4 == j).astype(np.float32)
                            for j in range(4)], -1) @ np.eye(4, 128,
                                                            dtype=np.float32))


def _pad_rows(x, n):
    return jnp.pad(x, ((0, n - x.shape[0]), (0, 0)))


@jax.jit
def kernel(img_batch, features, img_sizes, og_sizes, rpn_conv_w, rpn_conv_b,
           rpn_cls_w, rpn_cls_b, rpn_bbox_w, rpn_bbox_b, fc1_w, fc1_b,
           fc2_w, fc2_b, cls_w, cls_b, bbox_w, bbox_b):
    f32 = jnp.float32

    fp = jnp.pad(features, ((0, 0), (0, 0), (1, 1), (1, 1)))
    pats = jnp.stack([fp[:, :, dy:dy + _H, dx:dx + _W]
                      for dy in range(3) for dx in range(3)], axis=2)
    X = pats.transpose(0, 3, 4, 1, 2).reshape(_B, _H * _W, _C * 9)
    X = jnp.pad(X, ((0, 0), (0, _HWPAD - _H * _W), (0, 0)))
    Wc = rpn_conv_w.transpose(1, 2, 3, 0).reshape(_C * 9, _C)
    bc = jnp.broadcast_to(rpn_conv_b[None, :], (8, _C))
    Wh = jnp.zeros((_C, 128), f32)
    Wh = Wh.at[:, 0:_A].set(rpn_cls_w.T)
    Wh = Wh.at[:, _A:_A + 4 * _A].set(rpn_bbox_w.T)
    bh = jnp.zeros((128,), f32)
    bh = bh.at[0:_A].set(rpn_cls_b).at[_A:_A + 4 * _A].set(rpn_bbox_b)
    bh = jnp.broadcast_to(bh[None, :], (8, 128))

    ho = pl.pallas_call(
        _rpn_kernel,
        grid=(_B, _HWPAD // 256),
        in_specs=[
            pl.BlockSpec((1, 256, _C * 9), lambda b, i: (b, i, 0)),
            pl.BlockSpec((_C * 9, _C), lambda b, i: (0, 0)),
            pl.BlockSpec((8, _C), lambda b, i: (0, 0)),
            pl.BlockSpec((_C, 128), lambda b, i: (0, 0)),
            pl.BlockSpec((8, 128), lambda b, i: (0, 0)),
        ],
        out_specs=pl.BlockSpec((1, 256, 128), lambda b, i: (b, i, 0)),
        out_shape=jax.ShapeDtypeStruct((_B, _HWPAD, 128), f32),
    )(X, Wc, bc, Wh, bh)

    hw = ho[:, :_H * _W, :]
    obj = hw[:, :, 0:_A].transpose(0, 2, 1).reshape(_B, _NANCH)
    obj = jnp.pad(obj, ((0, 0), (0, _NPAD - _NANCH)),
                  constant_values=-1e30).reshape(_B, _NROW, 128)
    dl = hw[:, :, _A:_A + 4 * _A].reshape(_B, _H * _W, _A, 4)
    dl = dl.transpose(0, 2, 1, 3).reshape(_B, _NANCH, 4)
    dl = jnp.pad(dl, ((0, 0), (0, _NPAD - _NANCH), (0, 0)))
    dl = dl.transpose(0, 2, 1).reshape(_B, 4, _NROW, 128)

    pb = pl.pallas_call(
        _nms1_kernel,
        grid=(_B,),
        in_specs=[
            pl.BlockSpec((1, _NROW, 128), lambda b: (b, 0, 0)),
            pl.BlockSpec((1, 4, _NROW, 128), lambda b: (b, 0, 0, 0)),
            pl.BlockSpec((4, _NROW, 128), lambda b: (0, 0, 0)),
        ],
        out_specs=pl.BlockSpec((1, _NB, 128), lambda b: (b, 0, 0)),
        out_shape=jax.ShapeDtypeStruct((_B, _NB, 128), f32),
    )(obj, dl, _ANCHORS)

    F = features.transpose(0, 2, 3, 1).reshape(_B, _H * _W, _C)
    F = jnp.pad(F, ((0, 0), (0, _HWPAD - _H * _W), (0, 0)))
    W1p = fc1_w.reshape(_C, _POOL * _POOL, 1024).transpose(1, 0, 2)
    W1p = W1p.astype(jnp.bfloat16).astype(f32)
    b1 = jnp.broadcast_to(fc1_b[None, :], (8, 1024))

    h1 = pl.pallas_call(
        _roi_fc1_kernel,
        grid=(_B, _POOL * _POOL),
        in_specs=[
            pl.BlockSpec((1, _NB, 128), lambda b, k: (b, 0, 0)),
            pl.BlockSpec((1, _HWPAD, _C), lambda b, k: (b, 0, 0)),
            pl.BlockSpec((1, _C, 1024), lambda b, k: (k, 0, 0)),
            pl.BlockSpec((8, 1024), lambda b, k: (0, 0)),
            pl.BlockSpec((8, _HWPAD), lambda b, k: (0, 0)),
            pl.BlockSpec((8, _HWPAD), lambda b, k: (0, 0)),
        ],
        out_specs=pl.BlockSpec((1, _NB, 1024), lambda b, k: (b, 0, 0)),
        out_shape=jax.ShapeDtypeStruct((_B, _NB, 1024), f32),
    )(pb, F, W1p, b1, _COLY, _COLX)

    b2 = jnp.broadcast_to(fc2_b[None, :], (8, 1024))
    W2 = fc2_w.astype(jnp.bfloat16).astype(f32)
    Wcl = jnp.zeros((1024, 128), f32).at[:, :_NCLS].set(cls_w)
    Wcl = Wcl.astype(jnp.bfloat16).astype(f32)
    bcl = jnp.full((128,), -1e30, f32).at[:_NCLS].set(cls_b)
    bcl = jnp.broadcast_to(bcl[None, :], (8, 128))
    Wbb = jnp.zeros((1024, 384), f32).at[:, :4 * _NCLS].set(bbox_w)
    Wbb = Wbb.astype(jnp.bfloat16).astype(f32)
    bbb = jnp.zeros((384,), f32).at[:4 * _NCLS].set(bbox_b)
    bbb = jnp.broadcast_to(bbb[None, :], (8, 384))
    ratio = og_sizes.astype(f32) / float(_IMG)
    srow = jnp.zeros((_B, 8, 128), f32)
    srow = srow.at[:, 0, 0].set(ratio[:, 1]).at[:, 0, 1].set(ratio[:, 0])
    srow = srow.at[:, 0, 2].set(ratio[:, 1]).at[:, 0, 3].set(ratio[:, 0])
    srow = srow.at[:, 0, 4].set(1.0)

    out = pl.pallas_call(
        _head_kernel,
        grid=(_B,),
        in_specs=[
            pl.BlockSpec((1, _NB, 1024), lambda b: (b, 0, 0)),
            pl.BlockSpec((1024, 1024), lambda b: (0, 0)),
            pl.BlockSpec((8, 1024), lambda b: (0, 0)),
            pl.BlockSpec((1024, 128), lambda b: (0, 0)),
            pl.BlockSpec((8, 128), lambda b: (0, 0)),
            pl.BlockSpec((1024, 384), lambda b: (0, 0)),
            pl.BlockSpec((8, 384), lambda b: (0, 0)),
            pl.BlockSpec((384, 128), lambda b: (0, 0)),
            pl.BlockSpec((1, _NB, 128), lambda b: (b, 0, 0)),
            pl.BlockSpec((1, 8, 128), lambda b: (b, 0, 0)),
        ],
        out_specs=pl.BlockSpec((1, 104, 128), lambda b: (b, 0, 0)),
        out_shape=jax.ShapeDtypeStruct((_B, 104, 128), f32),
    )(h1, W2, b2, Wcl, bcl, Wbb, bbb, _RC, pb, srow)

    return out[:, :_DETS, :5]
